# baseline (speedup 1.0000x reference)
import functools

import jax
import jax.numpy as jnp
from jax import lax
from jax.experimental import pallas as pl
from jax.experimental.pallas import tpu as pltpu
from jax.experimental.pallas import tpu_sc as plsc

B = 4096
L = 200
EMB = 64
H1 = 128
NCLS = 100

NC = 2
NS = 16
NW = NC * NS
RPW = B // NW
NBUF = 4
G1 = 104


def _count_nonzero(idx_row):
    cnt = jnp.zeros((16,), jnp.int32)
    for c in range(L // 16):
        cnt = cnt + plsc.all_reduce_population_count(idx_row[pl.ds(c * 16, 16)] != 0)
    chunk = idx_row[pl.ds(L - 16, 16)]
    lane = lax.iota(jnp.int32, 16)
    cnt = cnt + plsc.all_reduce_population_count((chunk != 0) & (lane >= 8))
    return cnt


def _pool_body(x_hbm, emb_hbm, avg_hbm, idx_v, rows_v, out_v, sems):
    wid = lax.axis_index("s") * NC + lax.axis_index("c")
    base_row = wid * RPW

    pltpu.sync_copy(x_hbm.at[pl.ds(base_row, RPW), :], idx_v)
    idx2 = idx_v

    def gather_descs(r, k):
        d1 = pltpu.make_async_copy(
            emb_hbm.at[idx2.at[r, pl.ds(0, G1)]],
            rows_v.at[k, pl.ds(0, G1)], sems.at[k])
        d2 = pltpu.make_async_copy(
            emb_hbm.at[idx2.at[r, pl.ds(G1, L - G1)]],
            rows_v.at[k, pl.ds(G1, L - G1)], sems.at[k])
        return d1, d2

    def issue(r, k):
        d1, d2 = gather_descs(r, k)
        d1.start()
        d2.start()

    for k in range(NBUF - 1):
        issue(k, k)

    def row_step(r, k):
        d1, d2 = gather_descs(r, k)
        d1.wait()
        d2.wait()

        @pl.when(r + NBUF - 1 < RPW)
        def _():
            issue(r + NBUF - 1, (k + NBUF - 1) % NBUF)

        inv = 1.0 / jnp.maximum(_count_nonzero(idx2.at[r]).astype(jnp.float32),
                                1e-6)

        z = jnp.zeros((16,), jnp.float32)

        def sum_body(j, accs):
            accs = list(accs)
            for u in range(8):
                row = j * 8 + u
                s = 4 * (u % 2)
                for c in range(4):
                    accs[s + c] = accs[s + c] + rows_v[k, row, pl.ds(c * 16, 16)]
            return tuple(accs)

        a = lax.fori_loop(0, L // 8, sum_body, (z,) * 8)
        for c in range(4):
            out_v[r, pl.ds(c * 16, 16)] = (a[c] + a[4 + c]) * inv

    def loop_body(m, carry):
        for k in range(NBUF):
            row_step(m * NBUF + k, k)
        return carry

    lax.fori_loop(0, RPW // NBUF, loop_body, 0)
    pltpu.sync_copy(out_v, avg_hbm.at[pl.ds(base_row, RPW)])


@jax.jit
def _pool(x_flat, embed):
    mesh = plsc.VectorSubcoreMesh(core_axis_name="c", subcore_axis_name="s")
    return pl.kernel(
        _pool_body,
        mesh=mesh,
        compiler_params=pltpu.CompilerParams(
            needs_layout_passes=False, use_tc_tiling_on_sc=False),
        out_type=jax.ShapeDtypeStruct((B, EMB), jnp.float32),
        scratch_types=[
            pltpu.VMEM((RPW, L), jnp.int32),
            pltpu.VMEM((NBUF, L, EMB), jnp.float32),
            pltpu.VMEM((RPW, EMB), jnp.float32),
            pltpu.SemaphoreType.DMA((NBUF,)),
        ],
    )(x_flat, embed)


TT = 2048
VOCAB = 1000000


def _tr_body(in_ref, eye_ref, out_ref):
    out_ref[...] = jax.lax.dot_general(
        in_ref[...], eye_ref[...], (((0,), (0,)), ((), ())),
        preferred_element_type=jnp.float32)


def _transpose(embT):
    grid = (VOCAB + TT - 1) // TT
    eye = jnp.eye(EMB, dtype=jnp.float32)
    return pl.pallas_call(
        _tr_body,
        grid=(grid,),
        in_specs=[
            pl.BlockSpec((EMB, TT), lambda i: (0, i)),
            pl.BlockSpec((EMB, EMB), lambda i: (0, 0)),
        ],
        out_specs=pl.BlockSpec((TT, EMB), lambda i: (i, 0)),
        out_shape=jax.ShapeDtypeStruct((VOCAB, EMB), jnp.float32),
    )(embT, eye)


def _mlp_body(avg_ref, w1_ref, b1_ref, w2_ref, b2_ref, out_ref):
    h = jnp.dot(avg_ref[...], w1_ref[...], preferred_element_type=jnp.float32)
    h = jnp.maximum(h + b1_ref[...], 0.0)
    out_ref[...] = (
        jnp.dot(h, w2_ref[...], preferred_element_type=jnp.float32) + b2_ref[...])


def _mlp(avg, W1, b1, W2, b2):
    blk = 1024
    return pl.pallas_call(
        _mlp_body,
        grid=(B // blk,),
        in_specs=[
            pl.BlockSpec((blk, EMB), lambda i: (i, 0)),
            pl.BlockSpec((EMB, H1), lambda i: (0, 0)),
            pl.BlockSpec((1, H1), lambda i: (0, 0)),
            pl.BlockSpec((H1, NCLS), lambda i: (0, 0)),
            pl.BlockSpec((1, NCLS), lambda i: (0, 0)),
        ],
        out_specs=pl.BlockSpec((blk, NCLS), lambda i: (i, 0)),
        out_shape=jax.ShapeDtypeStruct((B, NCLS), jnp.float32),
    )(avg, W1, b1.reshape(1, H1), W2, b2.reshape(1, NCLS))


def kernel(x, embed, W1, b1, W2, b2):
    emb_rm = _transpose(embed.T)
    avg = _pool(x.astype(jnp.int32), emb_rm)
    return _mlp(avg, W1, b1, W2, b2)


# speedup vs baseline: 2.2467x; 2.2467x over previous
import functools

import jax
import jax.numpy as jnp
from jax import lax
from jax.experimental import pallas as pl
from jax.experimental.pallas import tpu as pltpu
from jax.experimental.pallas import tpu_sc as plsc

B = 4096
L = 200
EMB = 64
H1 = 128
NCLS = 100

NC = 2
NS = 16
NW = NC * NS
RPW = B // NW
NBUF = 4
G1 = 104


def _count_nonzero(idx_row):
    cnt = jnp.zeros((16,), jnp.int32)
    for c in range(L // 16):
        cnt = cnt + plsc.all_reduce_population_count(idx_row[pl.ds(c * 16, 16)] != 0)
    chunk = idx_row[pl.ds(L - 16, 16)]
    lane = lax.iota(jnp.int32, 16)
    cnt = cnt + plsc.all_reduce_population_count((chunk != 0) & (lane >= 8))
    return cnt


def _remap(u):
    half_sh = (TT // 2).bit_length() - 1
    return ((u & ~(TT - 1)) + ((u & (TT // 2 - 1)) << 1)
            + ((u >> half_sh) & 1))


def _pool_body(x_hbm, emb_hbm, avg_hbm, idx_v, rows_v, out_v, sems):
    wid = lax.axis_index("s") * NC + lax.axis_index("c")
    base_row = wid * RPW

    pltpu.sync_copy(x_hbm.at[pl.ds(base_row, RPW), :], idx_v)
    idx2 = idx_v

    lane = lax.iota(jnp.int32, 16)

    def remap_row(r, carry):
        for c in range(L // 16):
            chunk = idx2[r, pl.ds(c * 16, 16)]
            idx2[r, pl.ds(c * 16, 16)] = _remap(chunk)
        chunk = idx2[r, pl.ds(L - 16, 16)]
        idx2[r, pl.ds(L - 16, 16)] = jnp.where(lane >= 8, _remap(chunk), chunk)
        return carry

    lax.fori_loop(0, RPW, remap_row, 0)

    def gather_descs(r, k):
        d1 = pltpu.make_async_copy(
            emb_hbm.at[idx2.at[r, pl.ds(0, G1)]],
            rows_v.at[k, pl.ds(0, G1)], sems.at[k])
        d2 = pltpu.make_async_copy(
            emb_hbm.at[idx2.at[r, pl.ds(G1, L - G1)]],
            rows_v.at[k, pl.ds(G1, L - G1)], sems.at[k])
        return d1, d2

    def issue(r, k):
        d1, d2 = gather_descs(r, k)
        d1.start()
        d2.start()

    for k in range(NBUF - 1):
        issue(k, k)

    def row_step(r, k):
        d1, d2 = gather_descs(r, k)
        d1.wait()
        d2.wait()

        @pl.when(r + NBUF - 1 < RPW)
        def _():
            issue(r + NBUF - 1, (k + NBUF - 1) % NBUF)

        inv = 1.0 / jnp.maximum(_count_nonzero(idx2.at[r]).astype(jnp.float32),
                                1e-6)

        z = jnp.zeros((16,), jnp.float32)

        def sum_body(j, accs):
            accs = list(accs)
            for u in range(8):
                row = j * 8 + u
                s = 4 * (u % 2)
                for c in range(4):
                    accs[s + c] = accs[s + c] + rows_v[k, row, pl.ds(c * 16, 16)]
            return tuple(accs)

        a = lax.fori_loop(0, L // 8, sum_body, (z,) * 8)
        for c in range(4):
            out_v[r, pl.ds(c * 16, 16)] = (a[c] + a[4 + c]) * inv

    def loop_body(m, carry):
        for k in range(NBUF):
            row_step(m * NBUF + k, k)
        return carry

    lax.fori_loop(0, RPW // NBUF, loop_body, 0)
    pltpu.sync_copy(out_v, avg_hbm.at[pl.ds(base_row, RPW)])


@jax.jit
def _pool(x_flat, embed):
    mesh = plsc.VectorSubcoreMesh(core_axis_name="c", subcore_axis_name="s")
    return pl.kernel(
        _pool_body,
        mesh=mesh,
        compiler_params=pltpu.CompilerParams(
            needs_layout_passes=False, use_tc_tiling_on_sc=False),
        out_type=jax.ShapeDtypeStruct((B, EMB), jnp.float32),
        scratch_types=[
            pltpu.VMEM((RPW, L), jnp.int32),
            pltpu.VMEM((NBUF, L, EMB), jnp.float32),
            pltpu.VMEM((RPW, EMB), jnp.float32),
            pltpu.SemaphoreType.DMA((NBUF,)),
        ],
    )(x_flat, embed)


TT = 4096
VOCAB = 1000000
V_PAD = ((VOCAB + TT - 1) // TT) * TT


def _tr_body(in_ref, eye_ref, out_ref):
    t = jax.lax.dot_general(
        in_ref[...], eye_ref[...], (((0,), (0,)), ((), ())),
        preferred_element_type=jnp.float32)
    out_ref[:, :EMB] = t[: TT // 2]
    out_ref[:, EMB:] = t[TT // 2 :]


def _transpose(embT):
    grid = V_PAD // TT
    eye = jnp.eye(EMB, dtype=jnp.float32)
    return pl.pallas_call(
        _tr_body,
        grid=(grid,),
        in_specs=[
            pl.BlockSpec((EMB, TT), lambda i: (0, i)),
            pl.BlockSpec((EMB, EMB), lambda i: (0, 0)),
        ],
        out_specs=pl.BlockSpec((TT // 2, 2 * EMB), lambda i: (i, 0)),
        out_shape=jax.ShapeDtypeStruct((V_PAD // 2, 2 * EMB), jnp.float32),
    )(embT, eye)


def _mlp_body(avg_ref, w1_ref, b1_ref, w2_ref, b2_ref, out_ref):
    h = jnp.dot(avg_ref[...], w1_ref[...], preferred_element_type=jnp.float32)
    h = jnp.maximum(h + b1_ref[...], 0.0)
    out_ref[...] = (
        jnp.dot(h, w2_ref[...], preferred_element_type=jnp.float32) + b2_ref[...])


def _mlp(avg, W1, b1, W2, b2):
    blk = 1024
    return pl.pallas_call(
        _mlp_body,
        grid=(B // blk,),
        in_specs=[
            pl.BlockSpec((blk, EMB), lambda i: (i, 0)),
            pl.BlockSpec((EMB, H1), lambda i: (0, 0)),
            pl.BlockSpec((1, H1), lambda i: (0, 0)),
            pl.BlockSpec((H1, NCLS), lambda i: (0, 0)),
            pl.BlockSpec((1, NCLS), lambda i: (0, 0)),
        ],
        out_specs=pl.BlockSpec((blk, NCLS), lambda i: (i, 0)),
        out_shape=jax.ShapeDtypeStruct((B, NCLS), jnp.float32),
    )(avg, W1, b1.reshape(1, H1), W2, b2.reshape(1, NCLS))


def kernel(x, embed, W1, b1, W2, b2):
    emb_rm = _transpose(embed.T).reshape(V_PAD, EMB)
    avg = _pool(x.astype(jnp.int32), emb_rm)
    return _mlp(avg, W1, b1, W2, b2)


# speedup vs baseline: 2.2514x; 1.0021x over previous
import functools

import jax
import jax.numpy as jnp
from jax import lax
from jax.experimental import pallas as pl
from jax.experimental.pallas import tpu as pltpu
from jax.experimental.pallas import tpu_sc as plsc

B = 4096
L = 200
EMB = 64
H1 = 128
NCLS = 100

NC = 2
NS = 16
NW = NC * NS
RPW = B // NW
NBUF = 4
G1 = 104


def _count_nonzero(idx_row):
    cnt = jnp.zeros((16,), jnp.int32)
    for c in range(L // 16):
        cnt = cnt + plsc.all_reduce_population_count(idx_row[pl.ds(c * 16, 16)] != 0)
    chunk = idx_row[pl.ds(L - 16, 16)]
    lane = lax.iota(jnp.int32, 16)
    cnt = cnt + plsc.all_reduce_population_count((chunk != 0) & (lane >= 8))
    return cnt


def _remap(u):
    half_sh = (TT // 2).bit_length() - 1
    return ((u & ~(TT - 1)) + ((u & (TT // 2 - 1)) << 1)
            + ((u >> half_sh) & 1))


def _pool_body(x_hbm, emb_hbm, avg_hbm, idx_v, rows_v, out_v, sems):
    wid = lax.axis_index("s") * NC + lax.axis_index("c")
    base_row = wid * RPW

    pltpu.sync_copy(x_hbm.at[pl.ds(base_row, RPW), :], idx_v)
    idx2 = idx_v

    lane = lax.iota(jnp.int32, 16)

    def remap_row(r, carry):
        for c in range(L // 16):
            chunk = idx2[r, pl.ds(c * 16, 16)]
            idx2[r, pl.ds(c * 16, 16)] = _remap(chunk)
        chunk = idx2[r, pl.ds(L - 16, 16)]
        idx2[r, pl.ds(L - 16, 16)] = jnp.where(lane >= 8, _remap(chunk), chunk)
        return carry

    lax.fori_loop(0, RPW, remap_row, 0)

    def gather_descs(r, k):
        d1 = pltpu.make_async_copy(
            emb_hbm.at[idx2.at[r, pl.ds(0, G1)]],
            rows_v.at[k, pl.ds(0, G1)], sems.at[k])
        d2 = pltpu.make_async_copy(
            emb_hbm.at[idx2.at[r, pl.ds(G1, L - G1)]],
            rows_v.at[k, pl.ds(G1, L - G1)], sems.at[k])
        return d1, d2

    def issue(r, k):
        d1, d2 = gather_descs(r, k)
        d1.start()
        d2.start()

    for k in range(NBUF - 1):
        issue(k, k)

    def row_step(r, k):
        d1, d2 = gather_descs(r, k)
        d1.wait()
        d2.wait()

        @pl.when(r + NBUF - 1 < RPW)
        def _():
            issue(r + NBUF - 1, (k + NBUF - 1) % NBUF)

        inv = 1.0 / jnp.maximum(_count_nonzero(idx2.at[r]).astype(jnp.float32),
                                1e-6)

        z = jnp.zeros((16,), jnp.float32)

        def sum_body(j, accs):
            accs = list(accs)
            for u in range(8):
                row = j * 8 + u
                s = 4 * (u % 2)
                for c in range(4):
                    accs[s + c] = accs[s + c] + rows_v[k, row, pl.ds(c * 16, 16)]
            return tuple(accs)

        a = lax.fori_loop(0, L // 8, sum_body, (z,) * 8)
        for c in range(4):
            out_v[r, pl.ds(c * 16, 16)] = (a[c] + a[4 + c]) * inv

    def loop_body(m, carry):
        for k in range(NBUF):
            row_step(m * NBUF + k, k)
        return carry

    lax.fori_loop(0, RPW // NBUF, loop_body, 0)
    pltpu.sync_copy(out_v, avg_hbm.at[pl.ds(base_row, RPW)])


@jax.jit
def _pool(x_flat, embed):
    mesh = plsc.VectorSubcoreMesh(core_axis_name="c", subcore_axis_name="s")
    return pl.kernel(
        _pool_body,
        mesh=mesh,
        compiler_params=pltpu.CompilerParams(
            needs_layout_passes=False, use_tc_tiling_on_sc=False),
        out_type=jax.ShapeDtypeStruct((B, EMB), jnp.float32),
        scratch_types=[
            pltpu.VMEM((RPW, L), jnp.int32),
            pltpu.VMEM((NBUF, L, EMB), jnp.float32),
            pltpu.VMEM((RPW, EMB), jnp.float32),
            pltpu.SemaphoreType.DMA((NBUF,)),
        ],
    )(x_flat, embed)


TT = 4096
VOCAB = 1000000
V_PAD = ((VOCAB + TT - 1) // TT) * TT


def _tr_body(in_ref, eye_ref, out_ref):
    t = jax.lax.dot_general(
        in_ref[...], eye_ref[...], (((0,), (0,)), ((), ())),
        preferred_element_type=jnp.float32)
    out_ref[:, :EMB] = t[: TT // 2]
    out_ref[:, EMB:] = t[TT // 2 :]


def _transpose(embT):
    grid = V_PAD // TT
    eye = jnp.eye(EMB, dtype=jnp.float32)
    return pl.pallas_call(
        _tr_body,
        grid=(grid,),
        in_specs=[
            pl.BlockSpec((EMB, TT), lambda i: (0, i)),
            pl.BlockSpec((EMB, EMB), lambda i: (0, 0)),
        ],
        out_specs=pl.BlockSpec((TT // 2, 2 * EMB), lambda i: (i, 0)),
        out_shape=jax.ShapeDtypeStruct((V_PAD // 2, 2 * EMB), jnp.float32),
        compiler_params=pltpu.CompilerParams(
            dimension_semantics=("parallel",)),
    )(embT, eye)


def _mlp_body(avg_ref, w1_ref, b1_ref, w2_ref, b2_ref, out_ref):
    h = jnp.dot(avg_ref[...], w1_ref[...], preferred_element_type=jnp.float32)
    h = jnp.maximum(h + b1_ref[...], 0.0)
    out_ref[...] = (
        jnp.dot(h, w2_ref[...], preferred_element_type=jnp.float32) + b2_ref[...])


def _mlp(avg, W1, b1, W2, b2):
    blk = 1024
    return pl.pallas_call(
        _mlp_body,
        grid=(B // blk,),
        in_specs=[
            pl.BlockSpec((blk, EMB), lambda i: (i, 0)),
            pl.BlockSpec((EMB, H1), lambda i: (0, 0)),
            pl.BlockSpec((1, H1), lambda i: (0, 0)),
            pl.BlockSpec((H1, NCLS), lambda i: (0, 0)),
            pl.BlockSpec((1, NCLS), lambda i: (0, 0)),
        ],
        out_specs=pl.BlockSpec((blk, NCLS), lambda i: (i, 0)),
        out_shape=jax.ShapeDtypeStruct((B, NCLS), jnp.float32),
    )(avg, W1, b1.reshape(1, H1), W2, b2.reshape(1, NCLS))


def kernel(x, embed, W1, b1, W2, b2):
    emb_rm = _transpose(embed.T).reshape(V_PAD, EMB)
    avg = _pool(x.astype(jnp.int32), emb_rm)
    return _mlp(avg, W1, b1, W2, b2)


# speedup vs baseline: 2.6628x; 1.1827x over previous
import functools

import jax
import jax.numpy as jnp
from jax import lax
from jax.experimental import pallas as pl
from jax.experimental.pallas import tpu as pltpu
from jax.experimental.pallas import tpu_sc as plsc

B = 4096
L = 200
EMB = 64
H1 = 128
NCLS = 100

NC = 2
NS = 16
NW = NC * NS
RPW = B // NW
NBUF = 4
G1 = 104


def _count_nonzero(idx_row):
    cnt = jnp.zeros((16,), jnp.int32)
    for c in range(L // 16):
        cnt = cnt + plsc.all_reduce_population_count(idx_row[pl.ds(c * 16, 16)] != 0)
    chunk = idx_row[pl.ds(L - 16, 16)]
    lane = lax.iota(jnp.int32, 16)
    cnt = cnt + plsc.all_reduce_population_count((chunk != 0) & (lane >= 8))
    return cnt


def _remap(u):
    half_sh = (TT // 2).bit_length() - 1
    return ((u & ~(TT - 1)) + ((u & (TT // 2 - 1)) << 1)
            + ((u >> half_sh) & 1))


def _pool_body(x_hbm, emb_hbm, avg_hbm, idx_v, rows_v, out_v, sems):
    wid = lax.axis_index("s") * NC + lax.axis_index("c")
    base_row = wid * RPW

    pltpu.sync_copy(x_hbm.at[pl.ds(base_row, RPW), :], idx_v)
    idx2 = idx_v

    lane = lax.iota(jnp.int32, 16)

    def remap_row(r, carry):
        for c in range(L // 16):
            chunk = idx2[r, pl.ds(c * 16, 16)]
            idx2[r, pl.ds(c * 16, 16)] = _remap(chunk)
        chunk = idx2[r, pl.ds(L - 16, 16)]
        idx2[r, pl.ds(L - 16, 16)] = jnp.where(lane >= 8, _remap(chunk), chunk)
        return carry

    lax.fori_loop(0, RPW, remap_row, 0)

    def gather_descs(r, k):
        d1 = pltpu.make_async_copy(
            emb_hbm.at[idx2.at[r, pl.ds(0, G1)]],
            rows_v.at[k, pl.ds(0, G1)], sems.at[k])
        d2 = pltpu.make_async_copy(
            emb_hbm.at[idx2.at[r, pl.ds(G1, L - G1)]],
            rows_v.at[k, pl.ds(G1, L - G1)], sems.at[k])
        return d1, d2

    def issue(r, k):
        d1, d2 = gather_descs(r, k)
        d1.start()
        d2.start()

    for k in range(NBUF - 1):
        issue(k, k)

    def row_step(r, k):
        d1, d2 = gather_descs(r, k)
        d1.wait()
        d2.wait()

        @pl.when(r + NBUF - 1 < RPW)
        def _():
            issue(r + NBUF - 1, (k + NBUF - 1) % NBUF)

        inv = 1.0 / jnp.maximum(_count_nonzero(idx2.at[r]).astype(jnp.float32),
                                1e-6)

        z = jnp.zeros((16,), jnp.float32)

        def sum_body(j, accs):
            accs = list(accs)
            for u in range(8):
                row = j * 8 + u
                s = 4 * (u % 2)
                for c in range(4):
                    accs[s + c] = accs[s + c] + rows_v[k, row, pl.ds(c * 16, 16)]
            return tuple(accs)

        a = lax.fori_loop(0, L // 8, sum_body, (z,) * 8)
        for c in range(4):
            out_v[r, pl.ds(c * 16, 16)] = (a[c] + a[4 + c]) * inv

    def loop_body(m, carry):
        for k in range(NBUF):
            row_step(m * NBUF + k, k)
        return carry

    lax.fori_loop(0, RPW // NBUF, loop_body, 0)
    pltpu.sync_copy(out_v, avg_hbm.at[pl.ds(base_row, RPW)])


@jax.jit
def _pool(x_flat, embed):
    mesh = plsc.VectorSubcoreMesh(core_axis_name="c", subcore_axis_name="s")
    return pl.kernel(
        _pool_body,
        mesh=mesh,
        compiler_params=pltpu.CompilerParams(
            needs_layout_passes=False, use_tc_tiling_on_sc=False),
        out_type=jax.ShapeDtypeStruct((B, EMB), jnp.float32),
        scratch_types=[
            pltpu.VMEM((RPW, L), jnp.int32),
            pltpu.VMEM((NBUF, L, EMB), jnp.float32),
            pltpu.VMEM((RPW, EMB), jnp.float32),
            pltpu.SemaphoreType.DMA((NBUF,)),
        ],
    )(x_flat, embed)


TT = 8192
VOCAB = 1000000
V_PAD = ((VOCAB + TT - 1) // TT) * TT


def _tr_body(in_ref, eye_ref, out_ref):
    t = jax.lax.dot_general(
        in_ref[...], eye_ref[...], (((0,), (0,)), ((), ())),
        preferred_element_type=jnp.float32)
    out_ref[:, :EMB] = t[: TT // 2]
    out_ref[:, EMB:] = t[TT // 2 :]


def _transpose(embT):
    grid = V_PAD // TT
    eye = jnp.eye(EMB, dtype=jnp.float32)
    return pl.pallas_call(
        _tr_body,
        grid=(grid,),
        in_specs=[
            pl.BlockSpec((EMB, TT), lambda i: (0, i)),
            pl.BlockSpec((EMB, EMB), lambda i: (0, 0)),
        ],
        out_specs=pl.BlockSpec((TT // 2, 2 * EMB), lambda i: (i, 0)),
        out_shape=jax.ShapeDtypeStruct((V_PAD // 2, 2 * EMB), jnp.float32),
        compiler_params=pltpu.CompilerParams(
            dimension_semantics=("parallel",)),
    )(embT, eye)


def _mlp_body(avg_ref, w1_ref, b1_ref, w2_ref, b2_ref, out_ref):
    h = jnp.dot(avg_ref[...], w1_ref[...], preferred_element_type=jnp.float32)
    h = jnp.maximum(h + b1_ref[...], 0.0)
    out_ref[...] = (
        jnp.dot(h, w2_ref[...], preferred_element_type=jnp.float32) + b2_ref[...])


def _mlp(avg, W1, b1, W2, b2):
    blk = 1024
    return pl.pallas_call(
        _mlp_body,
        grid=(B // blk,),
        in_specs=[
            pl.BlockSpec((blk, EMB), lambda i: (i, 0)),
            pl.BlockSpec((EMB, H1), lambda i: (0, 0)),
            pl.BlockSpec((1, H1), lambda i: (0, 0)),
            pl.BlockSpec((H1, NCLS), lambda i: (0, 0)),
            pl.BlockSpec((1, NCLS), lambda i: (0, 0)),
        ],
        out_specs=pl.BlockSpec((blk, NCLS), lambda i: (i, 0)),
        out_shape=jax.ShapeDtypeStruct((B, NCLS), jnp.float32),
    )(avg, W1, b1.reshape(1, H1), W2, b2.reshape(1, NCLS))


def kernel(x, embed, W1, b1, W2, b2):
    emb_rm = _transpose(embed.T).reshape(V_PAD, EMB)
    avg = _pool(x.astype(jnp.int32), emb_rm)
    return _mlp(avg, W1, b1, W2, b2)


# speedup vs baseline: 2.9174x; 1.0956x over previous
import functools

import jax
import jax.numpy as jnp
from jax import lax
from jax.experimental import pallas as pl
from jax.experimental.pallas import tpu as pltpu
from jax.experimental.pallas import tpu_sc as plsc

B = 4096
L = 200
EMB = 64
H1 = 128
NCLS = 100

NC = 2
NS = 16
NW = NC * NS
RPW = B // NW
NBUF = 4
G1 = 104


def _count_nonzero(idx_row):
    cnt = jnp.zeros((16,), jnp.int32)
    for c in range(L // 16):
        cnt = cnt + plsc.all_reduce_population_count(idx_row[pl.ds(c * 16, 16)] != 0)
    chunk = idx_row[pl.ds(L - 16, 16)]
    lane = lax.iota(jnp.int32, 16)
    cnt = cnt + plsc.all_reduce_population_count((chunk != 0) & (lane >= 8))
    return cnt


def _remap(u):
    half_sh = (TT // 2).bit_length() - 1
    return ((u & ~(TT - 1)) + ((u & (TT // 2 - 1)) << 1)
            + ((u >> half_sh) & 1))


def _pool_body(x_hbm, emb_hbm, avg_hbm, idx_v, rows_v, out_v, sems):
    wid = lax.axis_index("s") * NC + lax.axis_index("c")
    base_row = wid * RPW

    pltpu.sync_copy(x_hbm.at[pl.ds(base_row, RPW), :], idx_v)
    idx2 = idx_v

    lane = lax.iota(jnp.int32, 16)

    def remap_row(r, carry):
        for c in range(L // 16):
            chunk = idx2[r, pl.ds(c * 16, 16)]
            idx2[r, pl.ds(c * 16, 16)] = _remap(chunk)
        chunk = idx2[r, pl.ds(L - 16, 16)]
        idx2[r, pl.ds(L - 16, 16)] = jnp.where(lane >= 8, _remap(chunk), chunk)
        return carry

    lax.fori_loop(0, RPW, remap_row, 0)

    def gather_descs(r, k):
        d1 = pltpu.make_async_copy(
            emb_hbm.at[idx2.at[r, pl.ds(0, G1)]],
            rows_v.at[k, pl.ds(0, G1)], sems.at[k])
        d2 = pltpu.make_async_copy(
            emb_hbm.at[idx2.at[r, pl.ds(G1, L - G1)]],
            rows_v.at[k, pl.ds(G1, L - G1)], sems.at[k])
        return d1, d2

    def issue(r, k):
        d1, d2 = gather_descs(r, k)
        d1.start()
        d2.start()

    for k in range(NBUF - 1):
        issue(k, k)

    def row_step(r, k):
        d1, d2 = gather_descs(r, k)
        d1.wait()
        d2.wait()

        @pl.when(r + NBUF - 1 < RPW)
        def _():
            issue(r + NBUF - 1, (k + NBUF - 1) % NBUF)

        inv = 1.0 / jnp.maximum(_count_nonzero(idx2.at[r]).astype(jnp.float32),
                                1e-6)

        z = jnp.zeros((16,), jnp.float32)

        def sum_body(j, accs):
            accs = list(accs)
            for u in range(8):
                row = j * 8 + u
                s = 4 * (u % 2)
                for c in range(4):
                    accs[s + c] = accs[s + c] + rows_v[k, row, pl.ds(c * 16, 16)]
            return tuple(accs)

        a = lax.fori_loop(0, L // 8, sum_body, (z,) * 8)
        for c in range(4):
            out_v[r, pl.ds(c * 16, 16)] = (a[c] + a[4 + c]) * inv

    def loop_body(m, carry):
        for k in range(NBUF):
            row_step(m * NBUF + k, k)
        return carry

    lax.fori_loop(0, RPW // NBUF, loop_body, 0)
    pltpu.sync_copy(out_v, avg_hbm.at[pl.ds(base_row, RPW)])


@jax.jit
def _pool(x_flat, embed):
    mesh = plsc.VectorSubcoreMesh(core_axis_name="c", subcore_axis_name="s")
    return pl.kernel(
        _pool_body,
        mesh=mesh,
        compiler_params=pltpu.CompilerParams(
            needs_layout_passes=False, use_tc_tiling_on_sc=False),
        out_type=jax.ShapeDtypeStruct((B, EMB), jnp.float32),
        scratch_types=[
            pltpu.VMEM((RPW, L), jnp.int32),
            pltpu.VMEM((NBUF, L, EMB), jnp.float32),
            pltpu.VMEM((RPW, EMB), jnp.float32),
            pltpu.SemaphoreType.DMA((NBUF,)),
        ],
    )(x_flat, embed)


TT = 16384
VOCAB = 1000000
V_PAD = ((VOCAB + TT - 1) // TT) * TT


def _tr_body(in_ref, eye_ref, out_ref):
    t = jax.lax.dot_general(
        in_ref[...], eye_ref[...], (((0,), (0,)), ((), ())),
        preferred_element_type=jnp.float32)
    out_ref[:, :EMB] = t[: TT // 2]
    out_ref[:, EMB:] = t[TT // 2 :]


def _transpose(embT):
    grid = V_PAD // TT
    eye = jnp.eye(EMB, dtype=jnp.float32)
    return pl.pallas_call(
        _tr_body,
        grid=(grid,),
        in_specs=[
            pl.BlockSpec((EMB, TT), lambda i: (0, i)),
            pl.BlockSpec((EMB, EMB), lambda i: (0, 0)),
        ],
        out_specs=pl.BlockSpec((TT // 2, 2 * EMB), lambda i: (i, 0)),
        out_shape=jax.ShapeDtypeStruct((V_PAD // 2, 2 * EMB), jnp.float32),
        compiler_params=pltpu.CompilerParams(
            dimension_semantics=("parallel",)),
    )(embT, eye)


def _mlp_body(avg_ref, w1_ref, b1_ref, w2_ref, b2_ref, out_ref):
    h = jnp.dot(avg_ref[...], w1_ref[...], preferred_element_type=jnp.float32)
    h = jnp.maximum(h + b1_ref[...], 0.0)
    out_ref[...] = (
        jnp.dot(h, w2_ref[...], preferred_element_type=jnp.float32) + b2_ref[...])


def _mlp(avg, W1, b1, W2, b2):
    blk = 1024
    return pl.pallas_call(
        _mlp_body,
        grid=(B // blk,),
        in_specs=[
            pl.BlockSpec((blk, EMB), lambda i: (i, 0)),
            pl.BlockSpec((EMB, H1), lambda i: (0, 0)),
            pl.BlockSpec((1, H1), lambda i: (0, 0)),
            pl.BlockSpec((H1, NCLS), lambda i: (0, 0)),
            pl.BlockSpec((1, NCLS), lambda i: (0, 0)),
        ],
        out_specs=pl.BlockSpec((blk, NCLS), lambda i: (i, 0)),
        out_shape=jax.ShapeDtypeStruct((B, NCLS), jnp.float32),
    )(avg, W1, b1.reshape(1, H1), W2, b2.reshape(1, NCLS))


def kernel(x, embed, W1, b1, W2, b2):
    emb_rm = _transpose(embed.T).reshape(V_PAD, EMB)
    avg = _pool(x.astype(jnp.int32), emb_rm)
    return _mlp(avg, W1, b1, W2, b2)


# speedup vs baseline: 3.0519x; 1.0461x over previous
import functools

import jax
import jax.numpy as jnp
from jax import lax
from jax.experimental import pallas as pl
from jax.experimental.pallas import tpu as pltpu
from jax.experimental.pallas import tpu_sc as plsc

B = 4096
L = 200
EMB = 64
H1 = 128
NCLS = 100

NC = 2
NS = 16
NW = NC * NS
RPW = B // NW
NBUF = 4
G1 = 104


def _count_nonzero(idx_row):
    cnt = jnp.zeros((16,), jnp.int32)
    for c in range(L // 16):
        cnt = cnt + plsc.all_reduce_population_count(idx_row[pl.ds(c * 16, 16)] != 0)
    chunk = idx_row[pl.ds(L - 16, 16)]
    lane = lax.iota(jnp.int32, 16)
    cnt = cnt + plsc.all_reduce_population_count((chunk != 0) & (lane >= 8))
    return cnt


def _remap(u):
    half_sh = (TT // 2).bit_length() - 1
    return ((u & ~(TT - 1)) + ((u & (TT // 2 - 1)) << 1)
            + ((u >> half_sh) & 1))


def _pool_body(x_hbm, emb_hbm, avg_hbm, idx_v, rows_v, out_v, sems):
    wid = lax.axis_index("s") * NC + lax.axis_index("c")
    base_row = wid * RPW

    pltpu.sync_copy(x_hbm.at[pl.ds(base_row, RPW), :], idx_v)
    idx2 = idx_v

    lane = lax.iota(jnp.int32, 16)

    def remap_row(r, carry):
        for c in range(L // 16):
            chunk = idx2[r, pl.ds(c * 16, 16)]
            idx2[r, pl.ds(c * 16, 16)] = _remap(chunk)
        chunk = idx2[r, pl.ds(L - 16, 16)]
        idx2[r, pl.ds(L - 16, 16)] = jnp.where(lane >= 8, _remap(chunk), chunk)
        return carry

    lax.fori_loop(0, RPW, remap_row, 0)

    def gather_descs(r, k):
        d1 = pltpu.make_async_copy(
            emb_hbm.at[idx2.at[r, pl.ds(0, G1)]],
            rows_v.at[k, pl.ds(0, G1)], sems.at[k])
        d2 = pltpu.make_async_copy(
            emb_hbm.at[idx2.at[r, pl.ds(G1, L - G1)]],
            rows_v.at[k, pl.ds(G1, L - G1)], sems.at[k])
        return d1, d2

    def issue(r, k):
        d1, d2 = gather_descs(r, k)
        d1.start()
        d2.start()

    for k in range(NBUF - 1):
        issue(k, k)

    def row_step(r, k):
        d1, d2 = gather_descs(r, k)
        d1.wait()
        d2.wait()

        @pl.when(r + NBUF - 1 < RPW)
        def _():
            issue(r + NBUF - 1, (k + NBUF - 1) % NBUF)

        inv = 1.0 / jnp.maximum(_count_nonzero(idx2.at[r]).astype(jnp.float32),
                                1e-6)

        z = jnp.zeros((16,), jnp.float32)

        def sum_body(j, accs):
            accs = list(accs)
            for u in range(8):
                row = j * 8 + u
                s = 4 * (u % 2)
                for c in range(4):
                    accs[s + c] = accs[s + c] + rows_v[k, row, pl.ds(c * 16, 16)]
            return tuple(accs)

        a = lax.fori_loop(0, L // 8, sum_body, (z,) * 8)
        for c in range(4):
            out_v[r, pl.ds(c * 16, 16)] = (a[c] + a[4 + c]) * inv

    def loop_body(m, carry):
        for k in range(NBUF):
            row_step(m * NBUF + k, k)
        return carry

    lax.fori_loop(0, RPW // NBUF, loop_body, 0)
    pltpu.sync_copy(out_v, avg_hbm.at[pl.ds(base_row, RPW)])


@jax.jit
def _pool(x_flat, embed):
    mesh = plsc.VectorSubcoreMesh(core_axis_name="c", subcore_axis_name="s")
    return pl.kernel(
        _pool_body,
        mesh=mesh,
        compiler_params=pltpu.CompilerParams(
            needs_layout_passes=False, use_tc_tiling_on_sc=False),
        out_type=jax.ShapeDtypeStruct((B, EMB), jnp.float32),
        scratch_types=[
            pltpu.VMEM((RPW, L), jnp.int32),
            pltpu.VMEM((NBUF, L, EMB), jnp.float32),
            pltpu.VMEM((RPW, EMB), jnp.float32),
            pltpu.SemaphoreType.DMA((NBUF,)),
        ],
    )(x_flat, embed)


TT = 32768
VOCAB = 1000000
V_PAD = ((VOCAB + TT - 1) // TT) * TT


def _tr_body(in_ref, eye_ref, out_ref):
    t = jax.lax.dot_general(
        in_ref[...], eye_ref[...], (((0,), (0,)), ((), ())),
        preferred_element_type=jnp.float32)
    out_ref[:, :EMB] = t[: TT // 2]
    out_ref[:, EMB:] = t[TT // 2 :]


def _transpose(embT):
    grid = V_PAD // TT
    eye = jnp.eye(EMB, dtype=jnp.float32)
    return pl.pallas_call(
        _tr_body,
        grid=(grid,),
        in_specs=[
            pl.BlockSpec((EMB, TT), lambda i: (0, i)),
            pl.BlockSpec((EMB, EMB), lambda i: (0, 0)),
        ],
        out_specs=pl.BlockSpec((TT // 2, 2 * EMB), lambda i: (i, 0)),
        out_shape=jax.ShapeDtypeStruct((V_PAD // 2, 2 * EMB), jnp.float32),
        compiler_params=pltpu.CompilerParams(
            dimension_semantics=("parallel",)),
    )(embT, eye)


def _mlp_body(avg_ref, w1_ref, b1_ref, w2_ref, b2_ref, out_ref):
    h = jnp.dot(avg_ref[...], w1_ref[...], preferred_element_type=jnp.float32)
    h = jnp.maximum(h + b1_ref[...], 0.0)
    out_ref[...] = (
        jnp.dot(h, w2_ref[...], preferred_element_type=jnp.float32) + b2_ref[...])


def _mlp(avg, W1, b1, W2, b2):
    blk = 1024
    return pl.pallas_call(
        _mlp_body,
        grid=(B // blk,),
        in_specs=[
            pl.BlockSpec((blk, EMB), lambda i: (i, 0)),
            pl.BlockSpec((EMB, H1), lambda i: (0, 0)),
            pl.BlockSpec((1, H1), lambda i: (0, 0)),
            pl.BlockSpec((H1, NCLS), lambda i: (0, 0)),
            pl.BlockSpec((1, NCLS), lambda i: (0, 0)),
        ],
        out_specs=pl.BlockSpec((blk, NCLS), lambda i: (i, 0)),
        out_shape=jax.ShapeDtypeStruct((B, NCLS), jnp.float32),
    )(avg, W1, b1.reshape(1, H1), W2, b2.reshape(1, NCLS))


def kernel(x, embed, W1, b1, W2, b2):
    emb_rm = _transpose(embed.T).reshape(V_PAD, EMB)
    avg = _pool(x.astype(jnp.int32), emb_rm)
    return _mlp(avg, W1, b1, W2, b2)
